# Initial kernel scaffold; baseline (speedup 1.0000x reference)
import jax
import jax.numpy as jnp
from jax.experimental import pallas as pl


def kernel(x, spatial_emb):
    raise NotImplementedError("write your pallas kernel here")


# speedup vs baseline: 2.5498x; 2.5498x over previous
import jax
import jax.numpy as jnp
from jax.experimental import pallas as pl
from jax.experimental.pallas import tpu as pltpu

GRID = 16


def _add_body(st_ref, x_ref, o_ref, row_ref):
    c, w = row_ref.shape
    cell_w = w // GRID

    @pl.when(pl.program_id(1) == 0)
    def _():
        tab = st_ref[0]
        g = jax.lax.broadcasted_iota(jnp.int32, (GRID, w), 0)
        j = jax.lax.broadcasted_iota(jnp.int32, (GRID, w), 1)
        sel = (j // cell_w == g).astype(jnp.float32)
        row_ref[...] = jnp.dot(tab, sel, preferred_element_type=jnp.float32)

    o_ref[...] = x_ref[...] + row_ref[...][:, None, None, :]


def kernel(x, spatial_emb):
    b, c, h, w = x.shape
    grid_size = spatial_emb.shape[1]
    cell_h = h // grid_size
    st = jnp.transpose(spatial_emb[0], (0, 2, 1))
    x4 = x.reshape(b * c, grid_size, cell_h, w)
    out = pl.pallas_call(
        _add_body,
        grid=(grid_size, b),
        in_specs=[
            pl.BlockSpec((1, c, grid_size), lambda gi, bb: (gi, 0, 0)),
            pl.BlockSpec((c, 1, cell_h, w), lambda gi, bb: (bb, gi, 0, 0)),
        ],
        out_specs=pl.BlockSpec((c, 1, cell_h, w), lambda gi, bb: (bb, gi, 0, 0)),
        out_shape=jax.ShapeDtypeStruct((b * c, grid_size, cell_h, w), x.dtype),
        scratch_shapes=[pltpu.VMEM((c, w), jnp.float32)],
    )(st, x4)
    return out.reshape(b, c, h, w)


# speedup vs baseline: 3.8604x; 1.5140x over previous
import jax
import jax.numpy as jnp
from jax.experimental import pallas as pl
from jax.experimental.pallas import tpu as pltpu


def kernel(x, spatial_emb):
    b, c, h, w = x.shape
    g = spatial_emb.shape[1]
    ch, cw = h // g, w // g
    band = ch * w
    k = 1
    while (k * band) % 128:
        k += 1
    lanes = k * band
    nrg = g // k
    kg = k * g
    rows_total = b * c * nrg

    tab = jnp.transpose(spatial_emb[0], (2, 0, 1)).reshape(c * nrg, kg)
    x2 = x.reshape(rows_total, lanes)

    BR = 128
    nblocks = rows_total // BR
    per_b = c * nrg // BR

    def body(tab_ref, x_ref, o_ref, sel_ref):
        @pl.when(pl.program_id(0) == 0)
        def _():
            l = jax.lax.broadcasted_iota(jnp.int32, (1, lanes), 1)
            code = (l // band) * g + (l % w) // cw
            gg = jax.lax.broadcasted_iota(jnp.int32, (kg, lanes), 0)
            sel_ref[...] = (code == gg).astype(jnp.float32)
        rows = jnp.dot(tab_ref[...], sel_ref[...],
                       preferred_element_type=jnp.float32)
        o_ref[...] = x_ref[...] + rows

    out = pl.pallas_call(
        body,
        grid=(nblocks,),
        in_specs=[
            pl.BlockSpec((BR, kg), lambda i: (i % per_b, 0)),
            pl.BlockSpec((BR, lanes), lambda i: (i, 0)),
        ],
        out_specs=pl.BlockSpec((BR, lanes), lambda i: (i, 0)),
        out_shape=jax.ShapeDtypeStruct((rows_total, lanes), x.dtype),
        scratch_shapes=[pltpu.VMEM((kg, lanes), jnp.float32)],
    )(tab, x2)
    return out.reshape(b, c, h, w)


# speedup vs baseline: 3.8945x; 1.0088x over previous
import jax
import jax.numpy as jnp
from jax.experimental import pallas as pl
from jax.experimental.pallas import tpu as pltpu


def kernel(x, spatial_emb):
    b, c, h, w = x.shape
    g = spatial_emb.shape[1]
    ch, cw = h // g, w // g
    band = ch * w
    k = 1
    while (k * band) % 128:
        k += 1
    lanes = k * band
    nrg = g // k
    kg = k * g
    rows_total = b * c * nrg

    tab = jnp.transpose(spatial_emb[0], (2, 0, 1)).reshape(c * nrg, kg)
    x2 = x.reshape(rows_total, lanes)

    BR = 256
    nblocks = rows_total // BR
    per_b = c * nrg // BR

    def body(tab_ref, x_ref, o_ref, sel_ref):
        @pl.when(pl.program_id(0) == 0)
        def _():
            l = jax.lax.broadcasted_iota(jnp.int32, (1, lanes), 1)
            code = (l // band) * g + (l % w) // cw
            gg = jax.lax.broadcasted_iota(jnp.int32, (kg, lanes), 0)
            sel_ref[...] = (code == gg).astype(jnp.float32)
        rows = jnp.dot(tab_ref[...], sel_ref[...],
                       preferred_element_type=jnp.float32)
        o_ref[...] = x_ref[...] + rows

    out = pl.pallas_call(
        body,
        grid=(nblocks,),
        in_specs=[
            pl.BlockSpec((BR, kg), lambda i: (i % per_b, 0)),
            pl.BlockSpec((BR, lanes), lambda i: (i, 0)),
        ],
        out_specs=pl.BlockSpec((BR, lanes), lambda i: (i, 0)),
        out_shape=jax.ShapeDtypeStruct((rows_total, lanes), x.dtype),
        scratch_shapes=[pltpu.VMEM((kg, lanes), jnp.float32)],
    )(tab, x2)
    return out.reshape(b, c, h, w)
